# Initial kernel scaffold; baseline (speedup 1.0000x reference)
import jax
import jax.numpy as jnp
from jax.experimental import pallas as pl


def kernel(x, adj, gc1_weight, gc2_weight):
    raise NotImplementedError("write your pallas kernel here")


# speedup vs baseline: 2.4345x; 2.4345x over previous
import jax
import jax.numpy as jnp
from jax.experimental import pallas as pl
from jax.experimental.pallas import tpu as pltpu


_PARALLEL = pltpu.CompilerParams(dimension_semantics=("parallel",))

_TM = 512


def _feat_kernel(x_ref, w1_ref, o_ref):
    o_ref[...] = jnp.dot(
        x_ref[...], w1_ref[...], preferred_element_type=jnp.float32
    ).astype(o_ref.dtype)


def _prop1_kernel(adj_ref, s1_ref, w2_ref, o_ref):
    a = adj_ref[...].astype(jnp.bfloat16)
    t = jnp.dot(a, s1_ref[...], preferred_element_type=jnp.float32)
    h = jnp.maximum(t, 0.0)
    o_ref[...] = jnp.dot(
        h, w2_ref[...], preferred_element_type=jnp.float32
    ).astype(o_ref.dtype)


def _prop2_kernel(adj_ref, s2_ref, o_ref):
    a = adj_ref[...].astype(jnp.bfloat16)
    o_ref[...] = jnp.dot(a, s2_ref[...], preferred_element_type=jnp.float32)


def kernel(x, adj, gc1_weight, gc2_weight):
    x = x.astype(jnp.float32)
    adj = adj.astype(jnp.float32)
    w1 = gc1_weight.astype(jnp.float32)
    w2 = gc2_weight.astype(jnp.float32)

    n, f = x.shape
    h1 = w1.shape[1]
    h2 = w2.shape[1]
    assert n % _TM == 0, n

    grid = (n // _TM,)

    s1 = pl.pallas_call(
        _feat_kernel,
        out_shape=jax.ShapeDtypeStruct((n, h1), jnp.bfloat16),
        grid=grid,
        in_specs=[
            pl.BlockSpec((_TM, f), lambda i: (i, 0)),
            pl.BlockSpec((f, h1), lambda i: (0, 0)),
        ],
        out_specs=pl.BlockSpec((_TM, h1), lambda i: (i, 0)),
        compiler_params=_PARALLEL,
    )(x, w1)

    s2 = pl.pallas_call(
        _prop1_kernel,
        out_shape=jax.ShapeDtypeStruct((n, h2), jnp.bfloat16),
        grid=grid,
        in_specs=[
            pl.BlockSpec((_TM, n), lambda i: (i, 0)),
            pl.BlockSpec((n, h1), lambda i: (0, 0)),
            pl.BlockSpec((h1, h2), lambda i: (0, 0)),
        ],
        out_specs=pl.BlockSpec((_TM, h2), lambda i: (i, 0)),
        compiler_params=_PARALLEL,
    )(adj, s1, w2)

    z = pl.pallas_call(
        _prop2_kernel,
        out_shape=jax.ShapeDtypeStruct((n, h2), jnp.float32),
        grid=grid,
        in_specs=[
            pl.BlockSpec((_TM, n), lambda i: (i, 0)),
            pl.BlockSpec((n, h2), lambda i: (0, 0)),
        ],
        out_specs=pl.BlockSpec((_TM, h2), lambda i: (i, 0)),
        compiler_params=_PARALLEL,
    )(adj, s2)

    return z, z, None
